# baseline (speedup 1.0000x reference)
import functools

import jax
import jax.numpy as jnp
from jax.experimental import pallas as pl
from jax.experimental.pallas import tpu as pltpu

_EPS = 1e-5


def _bn_coeffs(st_ref, g_ref, be_ref, count):
    s = st_ref[...]
    mean = s[0:1] * (1.0 / count)
    var = jnp.maximum(s[1:2] * (1.0 / count) - mean * mean, 0.0)
    scale = g_ref[...] * jax.lax.rsqrt(var + _EPS)
    shift = be_ref[...] - mean * scale
    return scale, shift


def _conv3x3(y, w_ref, m_ref, xp_ref, pall_ref, H, W, C):
    B, HW, _ = y.shape
    HW2 = (H + 2) * W
    M = B * HW2
    xp_ref[:, 0:W] = jnp.zeros((B, W, C), jnp.bfloat16)
    xp_ref[:, W:W + HW] = y
    xp_ref[:, W + HW:HW2] = jnp.zeros((B, W, C), jnp.bfloat16)
    d = xp_ref[...].reshape(M, C)
    zrow = jnp.zeros((1, C), jnp.bfloat16)
    m0 = m_ref[:, 0:C].reshape(1, HW2, C)
    m2 = m_ref[:, C:2 * C].reshape(1, HW2, C)
    y0 = jnp.concatenate([zrow, d[:M - 1]], axis=0).reshape(B, HW2, C) * m0
    y2 = jnp.concatenate([d[1:], zrow], axis=0).reshape(B, HW2, C) * m2
    pall_ref[:, 0:C] = y0.reshape(M, C)
    pall_ref[:, C:2 * C] = d
    pall_ref[:, 2 * C:3 * C] = y2.reshape(M, C)
    acc = jnp.dot(pall_ref[...], w_ref[...],
                  preferred_element_type=jnp.float32)
    a3 = acc.reshape(B, HW2, 3 * C)
    return (a3[:, 0:HW, 0:C]
            + a3[:, W:W + HW, C:2 * C]
            + a3[:, 2 * W:2 * W + HW, 2 * C:3 * C])


def _accum_stats(st_ref, z, first):
    B, HW, C = z.shape
    zf = z.reshape(B * HW, C)
    part = jnp.concatenate([jnp.sum(zf, axis=0, keepdims=True),
                            jnp.sum(zf * zf, axis=0, keepdims=True)], axis=0)

    @pl.when(first)
    def _():
        st_ref[...] = part

    @pl.when(jnp.logical_not(first))
    def _():
        st_ref[...] = st_ref[...] + part


def _body(x_ref, w1_ref, w2_ref, m_ref, g1_ref, be1_ref, g2_ref, be2_ref,
          o_ref, xp_ref, pall_ref, zs_ref, st1_ref, st2_ref, *, H, W, count):
    B, HW, C = x_ref.shape
    p = pl.program_id(0)
    n = pl.program_id(1)
    zsl = pl.ds(n * B, B)

    @pl.when(p == 0)
    def _phase0():
        y = x_ref[...].astype(jnp.bfloat16)
        z = _conv3x3(y, w1_ref, m_ref, xp_ref, pall_ref, H, W, C)
        _accum_stats(st1_ref, z, n == 0)
        zs_ref[zsl] = z.astype(jnp.bfloat16)

    @pl.when(p == 1)
    def _phase1():
        scale, shift = _bn_coeffs(st1_ref, g1_ref, be1_ref, count)
        y = jnp.maximum(zs_ref[zsl] * scale.astype(jnp.bfloat16)
                        + shift.astype(jnp.bfloat16), jnp.bfloat16(0))
        z = _conv3x3(y, w2_ref, m_ref, xp_ref, pall_ref, H, W, C)
        _accum_stats(st2_ref, z, n == 0)
        zs_ref[zsl] = z.astype(jnp.bfloat16)

    @pl.when(p == 2)
    def _phase2():
        scale, shift = _bn_coeffs(st2_ref, g2_ref, be2_ref, count)
        o_ref[...] = (zs_ref[zsl].astype(jnp.float32) * scale[None]
                      + shift[None] + x_ref[...])


def _pack_w(w):
    C = w.shape[2]
    return jnp.transpose(w, (1, 2, 0, 3)).reshape(3 * C, 3 * C).astype(
        jnp.bfloat16)


def kernel(x, w1, b1, g1, be1, w2, b2, g2, be2):
    N, H, W, C = x.shape
    HW, HW2 = H * W, (H + 2) * W
    count = float(N * H * W)
    xf = x.reshape(N, HW, C)
    w1p, w2p = _pack_w(w1), _pack_w(w2)
    wpos = jnp.arange(HW2, dtype=jnp.int32) % W
    masks = jnp.concatenate(
        [jnp.broadcast_to((wpos != 0)[:, None], (HW2, C)),
         jnp.broadcast_to((wpos != W - 1)[:, None], (HW2, C))],
        axis=1).astype(jnp.bfloat16)

    B = 4
    while N % B:
        B -= 1
    G = N // B

    x_spec = pl.BlockSpec((B, HW, C),
                          lambda p, n: (jnp.where(p == 1, 0, n), 0, 0))
    o_spec = pl.BlockSpec((B, HW, C),
                          lambda p, n: (jnp.where(p == 2, n, 0), 0, 0))
    w_spec = pl.BlockSpec((3 * C, 3 * C), lambda p, n: (0, 0))
    m_spec = pl.BlockSpec((HW2, 2 * C), lambda p, n: (0, 0))
    vec_spec = pl.BlockSpec((1, C), lambda p, n: (0, 0))

    out = pl.pallas_call(
        functools.partial(_body, H=H, W=W, count=count),
        grid=(3, G),
        in_specs=[x_spec, w_spec, w_spec, m_spec,
                  vec_spec, vec_spec, vec_spec, vec_spec],
        out_specs=o_spec,
        out_shape=jax.ShapeDtypeStruct((N, HW, C), jnp.float32),
        scratch_shapes=[
            pltpu.VMEM((B, HW2, C), jnp.bfloat16),
            pltpu.VMEM((B * HW2, 3 * C), jnp.bfloat16),
            pltpu.VMEM((N, HW, C), jnp.bfloat16),
            pltpu.VMEM((2, C), jnp.float32),
            pltpu.VMEM((2, C), jnp.float32),
        ],
        compiler_params=pltpu.CompilerParams(
            dimension_semantics=("arbitrary", "arbitrary"),
            vmem_limit_bytes=100 * 1024 * 1024),
    )(xf, w1p, w2p, masks, g1, be1, g2, be2)
    return out.reshape(N, H, W, C)


# speedup vs baseline: 1.0404x; 1.0404x over previous
import functools

import jax
import jax.numpy as jnp
from jax.experimental import pallas as pl
from jax.experimental.pallas import tpu as pltpu

_EPS = 1e-5


def _bn_coeffs(st_ref, g_ref, be_ref, count):
    s = st_ref[...]
    mean = s[0:1] * (1.0 / count)
    var = jnp.maximum(s[1:2] * (1.0 / count) - mean * mean, 0.0)
    scale = g_ref[...] * jax.lax.rsqrt(var + _EPS)
    shift = be_ref[...] - mean * scale
    return scale, shift


def _conv3x3(y, w_ref, m_ref, xp_ref, pall_ref, H, W, C):
    B, HW, _ = y.shape
    HW2 = (H + 2) * W
    M = B * HW2
    xp_ref[:, 0:W] = jnp.zeros((B, W, C), jnp.bfloat16)
    xp_ref[:, W:W + HW] = y
    xp_ref[:, W + HW:HW2] = jnp.zeros((B, W, C), jnp.bfloat16)
    d = xp_ref[...].reshape(M, C)
    zrow = jnp.zeros((1, C), jnp.bfloat16)
    m0 = m_ref[:, 0:C].reshape(1, HW2, C)
    m2 = m_ref[:, C:2 * C].reshape(1, HW2, C)
    y0 = jnp.concatenate([zrow, d[:M - 1]], axis=0).reshape(B, HW2, C) * m0
    y2 = jnp.concatenate([d[1:], zrow], axis=0).reshape(B, HW2, C) * m2
    pall_ref[:, 0:C] = y0.reshape(M, C)
    pall_ref[:, C:2 * C] = d
    pall_ref[:, 2 * C:3 * C] = y2.reshape(M, C)
    acc = jnp.dot(pall_ref[...], w_ref[...],
                  preferred_element_type=jnp.float32)
    a3 = acc.reshape(B, HW2, 3 * C)
    return (a3[:, 0:HW, 0:C]
            + a3[:, W:W + HW, C:2 * C]
            + a3[:, 2 * W:2 * W + HW, 2 * C:3 * C])


def _accum_stats(st_ref, z, first):
    B, HW, C = z.shape
    zf = z.reshape(B * HW, C)
    part = jnp.concatenate([jnp.sum(zf, axis=0, keepdims=True),
                            jnp.sum(zf * zf, axis=0, keepdims=True)], axis=0)

    @pl.when(first)
    def _():
        st_ref[...] = part

    @pl.when(jnp.logical_not(first))
    def _():
        st_ref[...] = st_ref[...] + part


def _body(x_ref, w1_ref, w2_ref, m_ref, g1_ref, be1_ref, g2_ref, be2_ref,
          o_ref, xp_ref, pall_ref, zs_ref, st1_ref, st2_ref, *, H, W, count):
    B, HW, C = x_ref.shape
    p = pl.program_id(0)
    n = pl.program_id(1)
    zsl = pl.ds(n * B, B)

    @pl.when(p == 0)
    def _phase0():
        y = x_ref[...].astype(jnp.bfloat16)
        z = _conv3x3(y, w1_ref, m_ref, xp_ref, pall_ref, H, W, C)
        _accum_stats(st1_ref, z, n == 0)
        zs_ref[zsl] = z.astype(jnp.bfloat16)

    @pl.when(p == 1)
    def _phase1():
        scale, shift = _bn_coeffs(st1_ref, g1_ref, be1_ref, count)
        y = jnp.maximum(zs_ref[zsl] * scale.astype(jnp.bfloat16)
                        + shift.astype(jnp.bfloat16), jnp.bfloat16(0))
        z = _conv3x3(y, w2_ref, m_ref, xp_ref, pall_ref, H, W, C)
        _accum_stats(st2_ref, z, n == 0)
        zs_ref[zsl] = z.astype(jnp.bfloat16)

    @pl.when(p == 2)
    def _phase2():
        scale, shift = _bn_coeffs(st2_ref, g2_ref, be2_ref, count)
        o_ref[...] = (zs_ref[zsl].astype(jnp.float32) * scale[None]
                      + shift[None] + x_ref[...])


def _pack_w(w):
    C = w.shape[2]
    return jnp.transpose(w, (1, 2, 0, 3)).reshape(3 * C, 3 * C).astype(
        jnp.bfloat16)


def kernel(x, w1, b1, g1, be1, w2, b2, g2, be2):
    N, H, W, C = x.shape
    HW, HW2 = H * W, (H + 2) * W
    count = float(N * H * W)
    xf = x.reshape(N, HW, C)
    w1p, w2p = _pack_w(w1), _pack_w(w2)
    wpos = jnp.arange(HW2, dtype=jnp.int32) % W
    masks = jnp.concatenate(
        [jnp.broadcast_to((wpos != 0)[:, None], (HW2, C)),
         jnp.broadcast_to((wpos != W - 1)[:, None], (HW2, C))],
        axis=1).astype(jnp.bfloat16)

    B = 8
    while N % B:
        B -= 1
    G = N // B

    x_spec = pl.BlockSpec((B, HW, C),
                          lambda p, n: (jnp.where(p == 1, 0, n), 0, 0))
    o_spec = pl.BlockSpec((B, HW, C),
                          lambda p, n: (jnp.where(p == 2, n, 0), 0, 0))
    w_spec = pl.BlockSpec((3 * C, 3 * C), lambda p, n: (0, 0))
    m_spec = pl.BlockSpec((HW2, 2 * C), lambda p, n: (0, 0))
    vec_spec = pl.BlockSpec((1, C), lambda p, n: (0, 0))

    out = pl.pallas_call(
        functools.partial(_body, H=H, W=W, count=count),
        grid=(3, G),
        in_specs=[x_spec, w_spec, w_spec, m_spec,
                  vec_spec, vec_spec, vec_spec, vec_spec],
        out_specs=o_spec,
        out_shape=jax.ShapeDtypeStruct((N, HW, C), jnp.float32),
        scratch_shapes=[
            pltpu.VMEM((B, HW2, C), jnp.bfloat16),
            pltpu.VMEM((B * HW2, 3 * C), jnp.bfloat16),
            pltpu.VMEM((N, HW, C), jnp.bfloat16),
            pltpu.VMEM((2, C), jnp.float32),
            pltpu.VMEM((2, C), jnp.float32),
        ],
        compiler_params=pltpu.CompilerParams(
            dimension_semantics=("arbitrary", "arbitrary"),
            vmem_limit_bytes=100 * 1024 * 1024),
    )(xf, w1p, w2p, masks, g1, be1, g2, be2)
    return out.reshape(N, H, W, C)


# speedup vs baseline: 1.0813x; 1.0394x over previous
import functools

import jax
import jax.numpy as jnp
from jax.experimental import pallas as pl
from jax.experimental.pallas import tpu as pltpu

_EPS = 1e-5


def _bn_coeffs(st_ref, g_ref, be_ref, count):
    s = st_ref[...]
    mean = s[0:1] * (1.0 / count)
    var = jnp.maximum(s[1:2] * (1.0 / count) - mean * mean, 0.0)
    scale = g_ref[...] * jax.lax.rsqrt(var + _EPS)
    shift = be_ref[...] - mean * scale
    return scale, shift


def _conv3x3(y, w_ref, m_ref, xp_ref, pall_ref, H, W, C):
    B, HW, _ = y.shape
    HW2 = (H + 2) * W
    M = B * HW2
    xp_ref[:, 0:W] = jnp.zeros((B, W, C), jnp.bfloat16)
    xp_ref[:, W:W + HW] = y
    xp_ref[:, W + HW:HW2] = jnp.zeros((B, W, C), jnp.bfloat16)
    d = xp_ref[...].reshape(M, C)
    zrow = jnp.zeros((1, C), jnp.bfloat16)
    m0 = m_ref[:, 0:C].reshape(1, HW2, C)
    m2 = m_ref[:, C:2 * C].reshape(1, HW2, C)
    y0 = jnp.concatenate([zrow, d[:M - 1]], axis=0).reshape(B, HW2, C) * m0
    y2 = jnp.concatenate([d[1:], zrow], axis=0).reshape(B, HW2, C) * m2
    pall_ref[:, 0:C] = y0.reshape(M, C)
    pall_ref[:, C:2 * C] = d
    pall_ref[:, 2 * C:3 * C] = y2.reshape(M, C)
    acc = jnp.dot(pall_ref[...], w_ref[...],
                  preferred_element_type=jnp.float32)
    a3 = acc.reshape(B, HW2, 3 * C)
    return (a3[:, 0:HW, 0:C]
            + a3[:, W:W + HW, C:2 * C]
            + a3[:, 2 * W:2 * W + HW, 2 * C:3 * C])


def _accum_stats(st_ref, z, first):
    B, HW, C = z.shape
    zf = z.reshape(B * HW, C)
    part = jnp.concatenate([jnp.sum(zf, axis=0, keepdims=True),
                            jnp.sum(zf * zf, axis=0, keepdims=True)], axis=0)

    @pl.when(first)
    def _():
        st_ref[...] = part

    @pl.when(jnp.logical_not(first))
    def _():
        st_ref[...] = st_ref[...] + part


def _body(x_ref, w1_ref, w2_ref, m_ref, g1_ref, be1_ref, g2_ref, be2_ref,
          o_ref, xp_ref, pall_ref, zs_ref, xb_ref, st1_ref, st2_ref,
          *, H, W, count):
    B, HW, C = x_ref.shape
    p = pl.program_id(0)
    n = pl.program_id(1)
    zsl = pl.ds(n * B, B)

    @pl.when(p == 0)
    def _phase0():
        y = x_ref[...].astype(jnp.bfloat16)
        xb_ref[zsl] = y
        z = _conv3x3(y, w1_ref, m_ref, xp_ref, pall_ref, H, W, C)
        _accum_stats(st1_ref, z, n == 0)
        zs_ref[zsl] = z.astype(jnp.bfloat16)

    @pl.when(p == 1)
    def _phase1():
        scale, shift = _bn_coeffs(st1_ref, g1_ref, be1_ref, count)
        y = jnp.maximum(zs_ref[zsl] * scale.astype(jnp.bfloat16)
                        + shift.astype(jnp.bfloat16), jnp.bfloat16(0))
        z = _conv3x3(y, w2_ref, m_ref, xp_ref, pall_ref, H, W, C)
        _accum_stats(st2_ref, z, n == 0)
        zs_ref[zsl] = z.astype(jnp.bfloat16)

    @pl.when(p == 2)
    def _phase2():
        scale, shift = _bn_coeffs(st2_ref, g2_ref, be2_ref, count)
        o_ref[...] = (zs_ref[zsl].astype(jnp.float32) * scale[None]
                      + shift[None] + xb_ref[zsl].astype(jnp.float32))


def _pack_w(w):
    C = w.shape[2]
    return jnp.transpose(w, (1, 2, 0, 3)).reshape(3 * C, 3 * C).astype(
        jnp.bfloat16)


def kernel(x, w1, b1, g1, be1, w2, b2, g2, be2):
    N, H, W, C = x.shape
    HW, HW2 = H * W, (H + 2) * W
    count = float(N * H * W)
    xf = x.reshape(N, HW, C)
    w1p, w2p = _pack_w(w1), _pack_w(w2)
    wpos = jnp.arange(HW2, dtype=jnp.int32) % W
    masks = jnp.concatenate(
        [jnp.broadcast_to((wpos != 0)[:, None], (HW2, C)),
         jnp.broadcast_to((wpos != W - 1)[:, None], (HW2, C))],
        axis=1).astype(jnp.bfloat16)

    B = 4
    while N % B:
        B -= 1
    G = N // B

    x_spec = pl.BlockSpec((B, HW, C),
                          lambda p, n: (jnp.where(p == 0, n, 0), 0, 0))
    o_spec = pl.BlockSpec((B, HW, C),
                          lambda p, n: (jnp.where(p == 2, n, 0), 0, 0))
    w_spec = pl.BlockSpec((3 * C, 3 * C), lambda p, n: (0, 0))
    m_spec = pl.BlockSpec((HW2, 2 * C), lambda p, n: (0, 0))
    vec_spec = pl.BlockSpec((1, C), lambda p, n: (0, 0))

    out = pl.pallas_call(
        functools.partial(_body, H=H, W=W, count=count),
        grid=(3, G),
        in_specs=[x_spec, w_spec, w_spec, m_spec,
                  vec_spec, vec_spec, vec_spec, vec_spec],
        out_specs=o_spec,
        out_shape=jax.ShapeDtypeStruct((N, HW, C), jnp.float32),
        scratch_shapes=[
            pltpu.VMEM((B, HW2, C), jnp.bfloat16),
            pltpu.VMEM((B * HW2, 3 * C), jnp.bfloat16),
            pltpu.VMEM((N, HW, C), jnp.bfloat16),
            pltpu.VMEM((N, HW, C), jnp.bfloat16),
            pltpu.VMEM((2, C), jnp.float32),
            pltpu.VMEM((2, C), jnp.float32),
        ],
        compiler_params=pltpu.CompilerParams(
            dimension_semantics=("arbitrary", "arbitrary"),
            vmem_limit_bytes=100 * 1024 * 1024),
    )(xf, w1p, w2p, masks, g1, be1, g2, be2)
    return out.reshape(N, H, W, C)


# speedup vs baseline: 1.1626x; 1.0751x over previous
import functools

import jax
import jax.numpy as jnp
from jax.experimental import pallas as pl
from jax.experimental.pallas import tpu as pltpu

_EPS = 1e-5


def _bn_coeffs(st_ref, g_ref, be_ref, count):
    s = st_ref[...]
    mean = s[0:1] * (1.0 / count)
    var = jnp.maximum(s[1:2] * (1.0 / count) - mean * mean, 0.0)
    scale = g_ref[...] * jax.lax.rsqrt(var + _EPS)
    shift = be_ref[...] - mean * scale
    return scale, shift


def _conv3x3(y, w_ref, m_ref, xp_ref, pall_ref, H, W, C):
    B, HW, _ = y.shape
    HW2 = (H + 2) * W
    M = B * HW2
    xp_ref[:, 0:W] = jnp.zeros((B, W, C), jnp.bfloat16)
    xp_ref[:, W:W + HW] = y
    xp_ref[:, W + HW:HW2] = jnp.zeros((B, W, C), jnp.bfloat16)
    d = xp_ref[...].reshape(M, C)
    zrow = jnp.zeros((1, C), jnp.bfloat16)
    m0 = m_ref[:, 0:C].reshape(1, HW2, C)
    m2 = m_ref[:, C:2 * C].reshape(1, HW2, C)
    y0 = jnp.concatenate([zrow, d[:M - 1]], axis=0).reshape(B, HW2, C) * m0
    y2 = jnp.concatenate([d[1:], zrow], axis=0).reshape(B, HW2, C) * m2
    pall_ref[:, 0:C] = y0.reshape(M, C)
    pall_ref[:, C:2 * C] = y2.reshape(M, C)
    acc = (jnp.dot(pall_ref[...], w_ref[0:2 * C],
                   preferred_element_type=jnp.float32)
           + jnp.dot(d, w_ref[2 * C:3 * C],
                     preferred_element_type=jnp.float32))
    a3 = acc.reshape(B, HW2, 3 * C)
    return (a3[:, 0:HW, 0:C]
            + a3[:, W:W + HW, C:2 * C]
            + a3[:, 2 * W:2 * W + HW, 2 * C:3 * C])


def _accum_stats(st_ref, z, first):
    B, HW, C = z.shape
    zf = z.reshape(B * HW, C)
    part = jnp.concatenate([jnp.sum(zf, axis=0, keepdims=True),
                            jnp.sum(zf * zf, axis=0, keepdims=True)], axis=0)

    @pl.when(first)
    def _():
        st_ref[...] = part

    @pl.when(jnp.logical_not(first))
    def _():
        st_ref[...] = st_ref[...] + part


def _body(x_ref, w1_ref, w2_ref, m_ref, g1_ref, be1_ref, g2_ref, be2_ref,
          o_ref, xp_ref, pall_ref, zs_ref, xb_ref, st1_ref, st2_ref,
          *, H, W, count):
    B, HW, C = x_ref.shape
    p = pl.program_id(0)
    n = pl.program_id(1)
    zsl = pl.ds(n * B, B)

    @pl.when(p == 0)
    def _phase0():
        y = x_ref[...].astype(jnp.bfloat16)
        xb_ref[zsl] = y
        z = _conv3x3(y, w1_ref, m_ref, xp_ref, pall_ref, H, W, C)
        _accum_stats(st1_ref, z, n == 0)
        zs_ref[zsl] = z.astype(jnp.bfloat16)

    @pl.when(p == 1)
    def _phase1():
        scale, shift = _bn_coeffs(st1_ref, g1_ref, be1_ref, count)
        y = jnp.maximum(zs_ref[zsl] * scale.astype(jnp.bfloat16)
                        + shift.astype(jnp.bfloat16), jnp.bfloat16(0))
        z = _conv3x3(y, w2_ref, m_ref, xp_ref, pall_ref, H, W, C)
        _accum_stats(st2_ref, z, n == 0)
        zs_ref[zsl] = z.astype(jnp.bfloat16)

    @pl.when(p == 2)
    def _phase2():
        scale, shift = _bn_coeffs(st2_ref, g2_ref, be2_ref, count)
        o_ref[...] = (zs_ref[zsl].astype(jnp.float32) * scale[None]
                      + shift[None] + xb_ref[zsl].astype(jnp.float32))


def _pack_w(w):
    C = w.shape[2]
    wt = jnp.transpose(w, (1, 2, 0, 3))
    wt = jnp.stack([wt[0], wt[2], wt[1]], axis=0)
    return wt.reshape(3 * C, 3 * C).astype(jnp.bfloat16)


def kernel(x, w1, b1, g1, be1, w2, b2, g2, be2):
    N, H, W, C = x.shape
    HW, HW2 = H * W, (H + 2) * W
    count = float(N * H * W)
    xf = x.reshape(N, HW, C)
    w1p, w2p = _pack_w(w1), _pack_w(w2)
    wpos = jnp.arange(HW2, dtype=jnp.int32) % W
    masks = jnp.concatenate(
        [jnp.broadcast_to((wpos != 0)[:, None], (HW2, C)),
         jnp.broadcast_to((wpos != W - 1)[:, None], (HW2, C))],
        axis=1).astype(jnp.bfloat16)

    B = 4
    while N % B:
        B -= 1
    G = N // B

    x_spec = pl.BlockSpec((B, HW, C),
                          lambda p, n: (jnp.where(p == 0, n, 0), 0, 0))
    o_spec = pl.BlockSpec((B, HW, C),
                          lambda p, n: (jnp.where(p == 2, n, 0), 0, 0))
    w_spec = pl.BlockSpec((3 * C, 3 * C), lambda p, n: (0, 0))
    m_spec = pl.BlockSpec((HW2, 2 * C), lambda p, n: (0, 0))
    vec_spec = pl.BlockSpec((1, C), lambda p, n: (0, 0))

    out = pl.pallas_call(
        functools.partial(_body, H=H, W=W, count=count),
        grid=(3, G),
        in_specs=[x_spec, w_spec, w_spec, m_spec,
                  vec_spec, vec_spec, vec_spec, vec_spec],
        out_specs=o_spec,
        out_shape=jax.ShapeDtypeStruct((N, HW, C), jnp.float32),
        scratch_shapes=[
            pltpu.VMEM((B, HW2, C), jnp.bfloat16),
            pltpu.VMEM((B * HW2, 2 * C), jnp.bfloat16),
            pltpu.VMEM((N, HW, C), jnp.bfloat16),
            pltpu.VMEM((N, HW, C), jnp.bfloat16),
            pltpu.VMEM((2, C), jnp.float32),
            pltpu.VMEM((2, C), jnp.float32),
        ],
        compiler_params=pltpu.CompilerParams(
            dimension_semantics=("arbitrary", "arbitrary"),
            vmem_limit_bytes=100 * 1024 * 1024),
    )(xf, w1p, w2p, masks, g1, be1, g2, be2)
    return out.reshape(N, H, W, C)
